# baseline (speedup 1.0000x reference)
import functools

import jax
import jax.numpy as jnp
from jax import lax
from jax.experimental import pallas as pl
from jax.experimental.pallas import tpu as pltpu
from jax.experimental.pallas import tpu_sc as plsc

B = 16384
VOCAB = 100000
H = 64
N_CAT = 3
N_CONT = 2
TRANS_DIM = 371

_NC = 2
_NS = 16
_NW = _NC * _NS
_LANES = 16
_BPW = B // _NW


def _sc_gather_body(table_hbm, type_hbm, idx_hbm, out_hbm,
                    t_v, i_v, g_v, rows_v, sem):
    wid = lax.axis_index("s") * _NC + lax.axis_index("c")
    base = wid * _BPW
    pltpu.sync_copy(type_hbm.at[pl.ds(base, _BPW)], t_v)
    pltpu.sync_copy(idx_hbm.at[pl.ds(base, _BPW)], i_v)
    for k in range(_BPW // _LANES):
        sl = pl.ds(k * _LANES, _LANES)
        t = t_v[sl]
        ix = i_v[sl]
        g_v[sl] = jnp.minimum(t, N_CAT - 1) * VOCAB + ix
    pltpu.async_copy(table_hbm.at[g_v], rows_v, sem).wait()
    pltpu.sync_copy(rows_v, out_hbm.at[pl.ds(base, _BPW)])


_sc_gather = functools.partial(
    pl.kernel,
    out_type=jax.ShapeDtypeStruct((B, H), jnp.float32),
    mesh=plsc.VectorSubcoreMesh(core_axis_name="c", subcore_axis_name="s"),
    scratch_types=[
        pltpu.VMEM((_BPW,), jnp.int32),
        pltpu.VMEM((_BPW,), jnp.int32),
        pltpu.VMEM((_BPW,), jnp.int32),
        pltpu.VMEM((_BPW, H), jnp.float32),
        pltpu.SemaphoreType.DMA,
    ],
    compiler_params=pltpu.CompilerParams(use_tc_tiling_on_sc=False),
)(_sc_gather_body)


_R = 512
_NBLK = B // _R


def _tc_dense_body(type_ref, contv_ref, tfT_ref,
                   contWT_ref, contbT_ref, transWT_ref, transb_ref, out_ref):
    t = type_ref[0]
    v = contv_ref[0]

    trans = lax.dot_general(
        transWT_ref[...], tfT_ref[...],
        dimension_numbers=(((0,), (0,)), ((), ())),
        preferred_element_type=jnp.float32,
    ) + transb_ref[...]

    w0 = contWT_ref[:, 0:1]
    w1 = contWT_ref[:, 1:2]
    b0 = contbT_ref[:, 0:1]
    b1 = contbT_ref[:, 1:2]
    is0 = t <= N_CAT
    cont = jnp.where(is0, v * w0 + b0, v * w1 + b1)

    is_cont = t >= N_CAT
    out_ref[...] = jnp.where(t < N_CAT + N_CONT, cont, trans)


_tc_dense = pl.pallas_call(
    _tc_dense_body,
    grid=(_NBLK,),
    in_specs=[
        pl.BlockSpec((1, 1, _R), lambda i: (i, 0, 0)),
        pl.BlockSpec((1, 1, _R), lambda i: (i, 0, 0)),
        pl.BlockSpec((TRANS_DIM, _R), lambda i: (0, i)),
        pl.BlockSpec((H, N_CONT), lambda i: (0, 0)),
        pl.BlockSpec((H, N_CONT), lambda i: (0, 0)),
        pl.BlockSpec((TRANS_DIM, H), lambda i: (0, 0)),
        pl.BlockSpec((H, 1), lambda i: (0, 0)),
    ],
    out_specs=pl.BlockSpec((H, _R), lambda i: (0, i)),
    out_shape=jax.ShapeDtypeStruct((H, B), jnp.float32),
)


def _tc_finalmerge_body(type_ref, catemb_ref, dense_ref, out_ref):
    t = type_ref[0]
    cat = jnp.transpose(catemb_ref[...])
    out_ref[...] = jnp.where(t < N_CAT, cat, dense_ref[...])


_tc_finalmerge = pl.pallas_call(
    _tc_finalmerge_body,
    grid=(_NBLK,),
    in_specs=[
        pl.BlockSpec((1, 1, _R), lambda i: (i, 0, 0)),
        pl.BlockSpec((_R, H), lambda i: (i, 0)),
        pl.BlockSpec((H, _R), lambda i: (0, i)),
    ],
    out_specs=pl.BlockSpec((H, _R), lambda i: (0, i)),
    out_shape=jax.ShapeDtypeStruct((H, B), jnp.float32),
)


def kernel(type_id, cat_index, cont_value, trans_feat, cat_tables,
           cont_W, cont_b, trans_W, trans_b):
    table_flat = cat_tables.reshape(N_CAT * VOCAB, H)
    cat_emb = _sc_gather(table_flat, type_id, cat_index)
    type3d = type_id.reshape(_NBLK, 1, _R)
    dense_t = _tc_dense(
        type3d,
        cont_value.reshape(_NBLK, 1, _R),
        trans_feat.T,
        cont_W.T,
        cont_b.T,
        trans_W.T,
        trans_b.reshape(H, 1),
    )
    out_t = _tc_finalmerge(type3d, cat_emb, dense_t)
    return out_t.T


# speedup vs baseline: 1.9533x; 1.9533x over previous
import functools

import jax
import jax.numpy as jnp
from jax import lax
from jax.experimental import pallas as pl
from jax.experimental.pallas import tpu as pltpu
from jax.experimental.pallas import tpu_sc as plsc

B = 16384
VOCAB = 100000
H = 64
N_CAT = 3
N_CONT = 2
TRANS_DIM = 371

_NC = 2
_NS = 16
_NW = _NC * _NS
_LANES = 16
_BPW = B // _NW


def _sc_gather_body(table_hbm, type_hbm, idx_hbm, out_hbm,
                    t_v, i_v, g_v, rows_v, sem):
    wid = lax.axis_index("s") * _NC + lax.axis_index("c")
    base = wid * _BPW
    pltpu.sync_copy(type_hbm.at[pl.ds(base, _BPW)], t_v)
    pltpu.sync_copy(idx_hbm.at[pl.ds(base, _BPW)], i_v)
    for k in range(_BPW // _LANES):
        sl = pl.ds(k * _LANES, _LANES)
        t = t_v[sl]
        ix = i_v[sl]
        g_v[sl] = jnp.minimum(t, N_CAT - 1) * VOCAB + ix

    def _issue(k, carry):
        gvec = g_v[pl.ds(k * _LANES, _LANES)]
        for i in range(_LANES):
            g = gvec[i]
            pltpu.make_async_copy(
                table_hbm.at[pl.ds(g, 1)],
                rows_v.at[pl.ds(k * _LANES + i, 1)],
                sem,
            ).start()
        return carry

    lax.fori_loop(0, _BPW // _LANES, _issue, 0)

    def _drain(j, carry):
        pltpu.make_async_copy(
            table_hbm.at[pl.ds(0, 1)], rows_v.at[pl.ds(j, 1)], sem
        ).wait()
        return carry

    lax.fori_loop(0, _BPW, _drain, 0)
    pltpu.sync_copy(rows_v, out_hbm.at[pl.ds(base, _BPW)])


_sc_gather = functools.partial(
    pl.kernel,
    out_type=jax.ShapeDtypeStruct((B, H), jnp.float32),
    mesh=plsc.VectorSubcoreMesh(core_axis_name="c", subcore_axis_name="s"),
    scratch_types=[
        pltpu.VMEM((_BPW,), jnp.int32),
        pltpu.VMEM((_BPW,), jnp.int32),
        pltpu.VMEM((_BPW,), jnp.int32),
        pltpu.VMEM((_BPW, H), jnp.float32),
        pltpu.SemaphoreType.DMA,
    ],
)(_sc_gather_body)


_R = 512
_NBLK = B // _R


def _tc_dense_body(type_ref, contv_ref, tfT_ref,
                   contWT_ref, contbT_ref, transWT_ref, transb_ref, out_ref):
    t = type_ref[0]
    v = contv_ref[0]

    trans = lax.dot_general(
        transWT_ref[...], tfT_ref[...],
        dimension_numbers=(((0,), (0,)), ((), ())),
        preferred_element_type=jnp.float32,
    ) + transb_ref[...]

    w0 = contWT_ref[:, 0:1]
    w1 = contWT_ref[:, 1:2]
    b0 = contbT_ref[:, 0:1]
    b1 = contbT_ref[:, 1:2]
    is0 = t <= N_CAT
    cont = jnp.where(is0, v * w0 + b0, v * w1 + b1)

    is_cont = t >= N_CAT
    out_ref[...] = jnp.where(t < N_CAT + N_CONT, cont, trans)


_tc_dense = pl.pallas_call(
    _tc_dense_body,
    grid=(_NBLK,),
    in_specs=[
        pl.BlockSpec((1, 1, _R), lambda i: (i, 0, 0)),
        pl.BlockSpec((1, 1, _R), lambda i: (i, 0, 0)),
        pl.BlockSpec((TRANS_DIM, _R), lambda i: (0, i)),
        pl.BlockSpec((H, N_CONT), lambda i: (0, 0)),
        pl.BlockSpec((H, N_CONT), lambda i: (0, 0)),
        pl.BlockSpec((TRANS_DIM, H), lambda i: (0, 0)),
        pl.BlockSpec((H, 1), lambda i: (0, 0)),
    ],
    out_specs=pl.BlockSpec((H, _R), lambda i: (0, i)),
    out_shape=jax.ShapeDtypeStruct((H, B), jnp.float32),
)


def _tc_finalmerge_body(type_ref, catemb_ref, dense_ref, out_ref):
    t = type_ref[0]
    cat = jnp.transpose(catemb_ref[...])
    out_ref[...] = jnp.where(t < N_CAT, cat, dense_ref[...])


_tc_finalmerge = pl.pallas_call(
    _tc_finalmerge_body,
    grid=(_NBLK,),
    in_specs=[
        pl.BlockSpec((1, 1, _R), lambda i: (i, 0, 0)),
        pl.BlockSpec((_R, H), lambda i: (i, 0)),
        pl.BlockSpec((H, _R), lambda i: (0, i)),
    ],
    out_specs=pl.BlockSpec((H, _R), lambda i: (0, i)),
    out_shape=jax.ShapeDtypeStruct((H, B), jnp.float32),
)


def kernel(type_id, cat_index, cont_value, trans_feat, cat_tables,
           cont_W, cont_b, trans_W, trans_b):
    table_flat = cat_tables.reshape(N_CAT * VOCAB, H)
    cat_emb = _sc_gather(table_flat, type_id, cat_index)
    type3d = type_id.reshape(_NBLK, 1, _R)
    dense_t = _tc_dense(
        type3d,
        cont_value.reshape(_NBLK, 1, _R),
        trans_feat.T,
        cont_W.T,
        cont_b.T,
        trans_W.T,
        trans_b.reshape(H, 1),
    )
    out_t = _tc_finalmerge(type3d, cat_emb, dense_t)
    return out_t.T


# speedup vs baseline: 2.1258x; 1.0883x over previous
import functools

import jax
import jax.numpy as jnp
from jax import lax
from jax.experimental import pallas as pl
from jax.experimental.pallas import tpu as pltpu
from jax.experimental.pallas import tpu_sc as plsc

B = 16384
VOCAB = 100000
H = 64
N_CAT = 3
N_CONT = 2
TRANS_DIM = 371

_NC = 2
_NS = 16
_NW = _NC * _NS
_LANES = 16
_BPW = B // _NW


def _sc_gather_body(table_hbm, type_hbm, idx_hbm, dense_hbm, out_hbm,
                    t_v, i_v, g_v, rows_v, sem):
    del dense_hbm
    wid = lax.axis_index("s") * _NC + lax.axis_index("c")
    base = wid * _BPW
    pltpu.sync_copy(type_hbm.at[pl.ds(base, _BPW)], t_v)
    pltpu.sync_copy(idx_hbm.at[pl.ds(base, _BPW)], i_v)
    for k in range(_BPW // _LANES):
        sl = pl.ds(k * _LANES, _LANES)
        t = t_v[sl]
        ix = i_v[sl]
        g_v[sl] = jnp.minimum(t, N_CAT - 1) * VOCAB + ix

    def _issue(k, carry):
        gvec = g_v[pl.ds(k * _LANES, _LANES)]
        for i in range(_LANES):
            g = gvec[i]
            pltpu.make_async_copy(
                table_hbm.at[pl.ds(g, 1)],
                rows_v.at[pl.ds(k * _LANES + i, 1)],
                sem,
            ).start()
        return carry

    lax.fori_loop(0, _BPW // _LANES, _issue, 0)

    def _drain(j, carry):
        pltpu.make_async_copy(
            table_hbm.at[pl.ds(0, 1)], rows_v.at[pl.ds(j, 1)], sem
        ).wait()
        return carry

    lax.fori_loop(0, _BPW, _drain, 0)
    pltpu.sync_copy(rows_v, out_hbm.at[pl.ds(base, _BPW)])


_sc_gather = functools.partial(
    pl.kernel,
    out_type=jax.ShapeDtypeStruct((B, H), jnp.float32),
    mesh=plsc.VectorSubcoreMesh(core_axis_name="c", subcore_axis_name="s"),
    scratch_types=[
        pltpu.VMEM((_BPW,), jnp.int32),
        pltpu.VMEM((_BPW,), jnp.int32),
        pltpu.VMEM((_BPW,), jnp.int32),
        pltpu.VMEM((_BPW, H), jnp.float32),
        pltpu.SemaphoreType.DMA,
    ],
)(_sc_gather_body)


_R = 512
_NBLK = B // _R


def _tc_dense_body(type_ref, contv_ref, tfT_ref,
                   contWT_ref, contbT_ref, transWT_ref, transb_ref, out_ref):
    t = type_ref[0]
    v = contv_ref[0]

    trans = lax.dot_general(
        transWT_ref[...], tfT_ref[...],
        dimension_numbers=(((0,), (0,)), ((), ())),
        preferred_element_type=jnp.float32,
    ) + transb_ref[...]

    w0 = contWT_ref[:, 0:1]
    w1 = contWT_ref[:, 1:2]
    b0 = contbT_ref[:, 0:1]
    b1 = contbT_ref[:, 1:2]
    is0 = t <= N_CAT
    cont = jnp.where(is0, v * w0 + b0, v * w1 + b1)

    is_cont = t >= N_CAT
    out_ref[...] = jnp.where(t < N_CAT + N_CONT, cont, trans)


_tc_dense = pl.pallas_call(
    _tc_dense_body,
    grid=(_NBLK,),
    in_specs=[
        pl.BlockSpec((1, 1, _R), lambda i: (i, 0, 0)),
        pl.BlockSpec((1, 1, _R), lambda i: (i, 0, 0)),
        pl.BlockSpec((TRANS_DIM, _R), lambda i: (0, i)),
        pl.BlockSpec((H, N_CONT), lambda i: (0, 0)),
        pl.BlockSpec((H, N_CONT), lambda i: (0, 0)),
        pl.BlockSpec((TRANS_DIM, H), lambda i: (0, 0)),
        pl.BlockSpec((H, 1), lambda i: (0, 0)),
    ],
    out_specs=pl.BlockSpec((H, _R), lambda i: (0, i)),
    out_shape=jax.ShapeDtypeStruct((H, B), jnp.float32),
)


def _tc_finalmerge_body(type_ref, catemb_ref, dense_ref, out_ref):
    t = type_ref[0]
    ident = (lax.broadcasted_iota(jnp.int32, (H, H), 0)
             == lax.broadcasted_iota(jnp.int32, (H, H), 1)).astype(jnp.float32)
    cat = lax.dot_general(
        ident, catemb_ref[...],
        dimension_numbers=(((1,), (1,)), ((), ())),
        preferred_element_type=jnp.float32,
    )
    out_ref[...] = jnp.where(t < N_CAT, cat, dense_ref[...])


_tc_finalmerge = pl.pallas_call(
    _tc_finalmerge_body,
    grid=(_NBLK,),
    in_specs=[
        pl.BlockSpec((1, 1, _R), lambda i: (i, 0, 0)),
        pl.BlockSpec((_R, H), lambda i: (i, 0)),
        pl.BlockSpec((H, _R), lambda i: (0, i)),
    ],
    out_specs=pl.BlockSpec((H, _R), lambda i: (0, i)),
    out_shape=jax.ShapeDtypeStruct((H, B), jnp.float32),
)


def kernel(type_id, cat_index, cont_value, trans_feat, cat_tables,
           cont_W, cont_b, trans_W, trans_b):
    table_flat = cat_tables.reshape(N_CAT * VOCAB, H)
    type3d = type_id.reshape(_NBLK, 1, _R)
    dense_t = _tc_dense(
        type3d,
        cont_value.reshape(_NBLK, 1, _R),
        trans_feat.T,
        cont_W.T,
        cont_b.T,
        trans_W.T,
        trans_b.reshape(H, 1),
    )
    cat_emb = _sc_gather(table_flat, type_id, cat_index, dense_t)
    out_t = _tc_finalmerge(type3d, cat_emb, dense_t)
    return out_t.T


# speedup vs baseline: 2.3727x; 1.1161x over previous
import functools

import jax
import jax.numpy as jnp
from jax import lax
from jax.experimental import pallas as pl
from jax.experimental.pallas import tpu as pltpu
from jax.experimental.pallas import tpu_sc as plsc

B = 16384
VOCAB = 100000
H = 64
N_CAT = 3
N_CONT = 2
TRANS_DIM = 371

_NC = 2
_NS = 16
_NW = _NC * _NS
_LANES = 16
_BPW = B // _NW


def _sc_gather_body(table_hbm, type_hbm, idx_hbm, dense_hbm, out_hbm,
                    t_v, i_v, g_v, rows_v, sem):
    del dense_hbm
    wid = lax.axis_index("s") * _NC + lax.axis_index("c")
    base = wid * _BPW
    pltpu.sync_copy(type_hbm.at[pl.ds(base, _BPW)], t_v)
    pltpu.sync_copy(idx_hbm.at[pl.ds(base, _BPW)], i_v)
    for k in range(_BPW // _LANES):
        sl = pl.ds(k * _LANES, _LANES)
        t = t_v[sl]
        ix = i_v[sl]
        g_v[sl] = jnp.minimum(t, N_CAT - 1) * VOCAB + ix

    def _issue(k, carry):
        gvec = g_v[pl.ds(k * _LANES, _LANES)]
        for i in range(_LANES):
            g = gvec[i]
            pltpu.make_async_copy(
                table_hbm.at[pl.ds(g, 1)],
                rows_v.at[pl.ds(k * _LANES + i, 1)],
                sem,
            ).start()
        return carry

    lax.fori_loop(0, _BPW // _LANES, _issue, 0)

    def _drain(j, carry):
        pltpu.make_async_copy(
            table_hbm.at[pl.ds(0, 1)], rows_v.at[pl.ds(j, 1)], sem
        ).wait()
        return carry

    lax.fori_loop(0, _BPW, _drain, 0)
    pltpu.sync_copy(rows_v, out_hbm.at[pl.ds(base, _BPW)])


_sc_gather = functools.partial(
    pl.kernel,
    out_type=jax.ShapeDtypeStruct((B, H), jnp.float32),
    mesh=plsc.VectorSubcoreMesh(core_axis_name="c", subcore_axis_name="s"),
    scratch_types=[
        pltpu.VMEM((_BPW,), jnp.int32),
        pltpu.VMEM((_BPW,), jnp.int32),
        pltpu.VMEM((_BPW,), jnp.int32),
        pltpu.VMEM((_BPW, H), jnp.float32),
        pltpu.SemaphoreType.DMA,
    ],
)(_sc_gather_body)


_R = 512
_NBLK = B // _R


def _tc_dense_body(type_ref, contv_ref, tfT_ref,
                   contWT_ref, contbT_ref, transWT_ref, transb_ref, out_ref):
    t = type_ref[0]
    v = contv_ref[0]

    trans = lax.dot_general(
        transWT_ref[...], tfT_ref[...],
        dimension_numbers=(((0,), (0,)), ((), ())),
        preferred_element_type=jnp.float32,
    ) + transb_ref[...]

    w0 = contWT_ref[:, 0:1]
    w1 = contWT_ref[:, 1:2]
    b0 = contbT_ref[:, 0:1]
    b1 = contbT_ref[:, 1:2]
    is0 = t <= N_CAT
    cont = jnp.where(is0, v * w0 + b0, v * w1 + b1)

    is_cont = t >= N_CAT
    out_ref[...] = jnp.where(t < N_CAT + N_CONT, cont, trans)


_tc_dense = pl.pallas_call(
    _tc_dense_body,
    grid=(_NBLK,),
    in_specs=[
        pl.BlockSpec((1, 1, _R), lambda i: (i, 0, 0)),
        pl.BlockSpec((1, 1, _R), lambda i: (i, 0, 0)),
        pl.BlockSpec((TRANS_DIM, _R), lambda i: (0, i)),
        pl.BlockSpec((H, N_CONT), lambda i: (0, 0)),
        pl.BlockSpec((H, N_CONT), lambda i: (0, 0)),
        pl.BlockSpec((TRANS_DIM, H), lambda i: (0, 0)),
        pl.BlockSpec((H, 1), lambda i: (0, 0)),
    ],
    out_specs=pl.BlockSpec((H, _R), lambda i: (0, i)),
    out_shape=jax.ShapeDtypeStruct((H, B), jnp.float32),
)


def _tc_finalmerge_body(type_ref, catemb_ref, dense_ref, out_ref):
    t = type_ref[0]
    ident = (lax.broadcasted_iota(jnp.int32, (H, H), 0)
             == lax.broadcasted_iota(jnp.int32, (H, H), 1)).astype(jnp.float32)
    cat = lax.dot_general(
        ident, catemb_ref[...],
        dimension_numbers=(((1,), (1,)), ((), ())),
        preferred_element_type=jnp.float32,
    )
    out_ref[...] = jnp.where(t < N_CAT, cat, dense_ref[...])


_RM = 2048
_NBLKM = B // _RM

_tc_finalmerge = pl.pallas_call(
    _tc_finalmerge_body,
    grid=(_NBLKM,),
    in_specs=[
        pl.BlockSpec((1, 1, _RM), lambda i: (i, 0, 0)),
        pl.BlockSpec((_RM, H), lambda i: (i, 0)),
        pl.BlockSpec((H, _RM), lambda i: (0, i)),
    ],
    out_specs=pl.BlockSpec((H, _RM), lambda i: (0, i)),
    out_shape=jax.ShapeDtypeStruct((H, B), jnp.float32),
)


def kernel(type_id, cat_index, cont_value, trans_feat, cat_tables,
           cont_W, cont_b, trans_W, trans_b):
    table_flat = cat_tables.reshape(N_CAT * VOCAB, H)
    type3d = type_id.reshape(_NBLK, 1, _R)
    dense_t = _tc_dense(
        type3d,
        cont_value.reshape(_NBLK, 1, _R),
        trans_feat.T,
        cont_W.T,
        cont_b.T,
        trans_W.T,
        trans_b.reshape(H, 1),
    )
    cat_emb = _sc_gather(table_flat, type_id, cat_index, dense_t)
    out_t = _tc_finalmerge(type_id.reshape(_NBLKM, 1, _RM), cat_emb, dense_t)
    return out_t.T
